# Initial kernel scaffold; baseline (speedup 1.0000x reference)
import jax
import jax.numpy as jnp
from jax.experimental import pallas as pl


def kernel(x, codebooks):
    raise NotImplementedError("write your pallas kernel here")


# speedup vs baseline: 1.4313x; 1.4313x over previous
import jax
import jax.numpy as jnp
from jax.experimental import pallas as pl

_BETA = 0.25


def _rvq_kernel(x_ref, cb_ref, xq_ref, idx_ref, loss_ref, dist_ref):
    num_q, n_e, e_dim = cb_ref.shape
    bb = x_ref.shape[0]
    r = x_ref[...]
    xq_acc = jnp.zeros_like(r)
    loss_acc = jnp.float32(0.0)
    idx_cols = []
    for q in range(num_q):
        cb = cb_ref[q]
        c2 = jnp.sum(cb * cb, axis=1)
        rs = jnp.sum(r * r, axis=1, keepdims=True)
        cross = jax.lax.dot_general(
            r.astype(jnp.bfloat16), cb.astype(jnp.bfloat16),
            (((1,), (1,)), ((), ())),
            preferred_element_type=jnp.float32)
        d = rs - 2.0 * cross + c2[None, :]
        dist_ref[:, q, :] = d
        idx = jnp.argmin(d, axis=1).astype(jnp.int32)
        idx_cols.append(idx)
        one_hot = (jax.lax.broadcasted_iota(jnp.int32, (bb, n_e), 1)
                   == idx[:, None]).astype(jnp.float32)
        xq_rows = jax.lax.dot_general(
            one_hot, cb, (((1,), (0,)), ((), ())),
            precision=jax.lax.Precision.HIGHEST,
            preferred_element_type=jnp.float32)
        diff = xq_rows - r
        loss_acc = loss_acc + jnp.sum(diff * diff)
        xq_acc = xq_acc + xq_rows
        r = r - xq_rows
    xq_ref[...] = xq_acc
    idx_ref[...] = jnp.stack(idx_cols, axis=1)

    step = pl.program_id(0)
    total_b = pl.num_programs(0) * bb
    prev = jnp.where(step == 0, jnp.zeros((1, 1), jnp.float32), loss_ref[...])
    total = prev + loss_acc
    scale = (1.0 + _BETA) / (total_b * e_dim * num_q)
    loss_ref[...] = jnp.where(step == pl.num_programs(0) - 1,
                              total * scale, total)


def kernel(x, codebooks):
    b, e_dim = x.shape
    num_q, n_e, _ = codebooks.shape
    bb = min(256, b)
    grid = b // bb
    out_shapes = (
        jax.ShapeDtypeStruct((b, e_dim), jnp.float32),
        jax.ShapeDtypeStruct((b, num_q), jnp.int32),
        jax.ShapeDtypeStruct((1, 1), jnp.float32),
        jax.ShapeDtypeStruct((b, num_q, n_e), jnp.float32),
    )
    in_specs = [
        pl.BlockSpec((bb, e_dim), lambda i: (i, 0)),
        pl.BlockSpec((num_q, n_e, e_dim), lambda i: (0, 0, 0)),
    ]
    out_specs = (
        pl.BlockSpec((bb, e_dim), lambda i: (i, 0)),
        pl.BlockSpec((bb, num_q), lambda i: (i, 0)),
        pl.BlockSpec((1, 1), lambda i: (0, 0)),
        pl.BlockSpec((bb, num_q, n_e), lambda i: (i, 0, 0)),
    )
    xq, idx, loss, dist = pl.pallas_call(
        _rvq_kernel,
        grid=(grid,),
        in_specs=in_specs,
        out_specs=out_specs,
        out_shape=out_shapes,
    )(x, codebooks)
    return xq, loss.reshape(()), idx, dist


# speedup vs baseline: 2.2873x; 1.5981x over previous
import jax
import jax.numpy as jnp
from jax.experimental import pallas as pl
from jax.experimental.pallas import tpu as pltpu

_BETA = 0.25


def _rvq_kernel(x_ref, cb_ref, xq_ref, idx_ref, loss_ref, dist_ref,
                c2_ref, cb1_ref, cb2_ref, cb3_ref):
    num_q, n_e, e_dim = cb_ref.shape
    bb = x_ref.shape[0]
    f32 = jnp.float32
    bf16 = jnp.bfloat16

    @pl.when(pl.program_id(0) == 0)
    def _precompute():
        for q in range(num_q):
            cb = cb_ref[q]
            c1 = cb.astype(bf16)
            e1 = cb - c1.astype(f32)
            c2s = e1.astype(bf16)
            e2 = e1 - c2s.astype(f32)
            cb1_ref[q] = c1
            cb2_ref[q] = c2s
            cb3_ref[q] = e2.astype(bf16)
            c2_ref[q] = jnp.sum(cb * cb, axis=1)[None, :]

    r = x_ref[...]
    xq_acc = jnp.zeros_like(r)
    loss_acc = jnp.float32(0.0)
    idx_cols = []
    for q in range(num_q):
        cb1 = cb1_ref[q]
        rs = jnp.sum(r * r, axis=1, keepdims=True)
        cross2 = jax.lax.dot_general(
            (r + r).astype(bf16), cb1, (((1,), (1,)), ((), ())),
            preferred_element_type=f32)
        d = rs - cross2 + c2_ref[q]
        dist_ref[:, q, :] = d
        idx = jnp.argmin(d, axis=1).astype(jnp.int32)
        idx_cols.append(idx)
        one_hot = (jax.lax.broadcasted_iota(jnp.int32, (bb, n_e), 1)
                   == idx[:, None]).astype(f32).astype(bf16)
        gat = lambda c_part: jax.lax.dot_general(
            one_hot, c_part, (((1,), (0,)), ((), ())),
            preferred_element_type=f32)
        xq_rows = (gat(cb1) + gat(cb2_ref[q])) + gat(cb3_ref[q])
        diff = xq_rows - r
        loss_acc = loss_acc + jnp.sum(diff * diff)
        xq_acc = xq_acc + xq_rows
        r = r - xq_rows
    xq_ref[...] = xq_acc
    idx_ref[...] = jnp.stack(idx_cols, axis=1)

    step = pl.program_id(0)
    total_b = pl.num_programs(0) * bb
    prev = jnp.where(step == 0, jnp.zeros((1, 1), f32), loss_ref[...])
    total = prev + loss_acc
    scale = (1.0 + _BETA) / (total_b * e_dim * num_q)
    loss_ref[...] = jnp.where(step == pl.num_programs(0) - 1,
                              total * scale, total)


def kernel(x, codebooks):
    b, e_dim = x.shape
    num_q, n_e, _ = codebooks.shape
    bb = min(256, b)
    grid = b // bb
    out_shapes = (
        jax.ShapeDtypeStruct((b, e_dim), jnp.float32),
        jax.ShapeDtypeStruct((b, num_q), jnp.int32),
        jax.ShapeDtypeStruct((1, 1), jnp.float32),
        jax.ShapeDtypeStruct((b, num_q, n_e), jnp.float32),
    )
    in_specs = [
        pl.BlockSpec((bb, e_dim), lambda i: (i, 0)),
        pl.BlockSpec((num_q, n_e, e_dim), lambda i: (0, 0, 0)),
    ]
    out_specs = (
        pl.BlockSpec((bb, e_dim), lambda i: (i, 0)),
        pl.BlockSpec((bb, num_q), lambda i: (i, 0)),
        pl.BlockSpec((1, 1), lambda i: (0, 0)),
        pl.BlockSpec((bb, num_q, n_e), lambda i: (i, 0, 0)),
    )
    scratch_shapes = [
        pltpu.VMEM((num_q, 1, n_e), jnp.float32),
        pltpu.VMEM((num_q, n_e, e_dim), jnp.bfloat16),
        pltpu.VMEM((num_q, n_e, e_dim), jnp.bfloat16),
        pltpu.VMEM((num_q, n_e, e_dim), jnp.bfloat16),
    ]
    xq, idx, loss, dist = pl.pallas_call(
        _rvq_kernel,
        grid=(grid,),
        in_specs=in_specs,
        out_specs=out_specs,
        out_shape=out_shapes,
        scratch_shapes=scratch_shapes,
    )(x, codebooks)
    return xq, loss.reshape(()), idx, dist
